# baseline (speedup 1.0000x reference)
import functools
import numpy as np
import jax
import jax.numpy as jnp
from jax import lax
from jax.experimental import pallas as pl
from jax.experimental.pallas import tpu as pltpu
from jax.experimental.pallas import tpu_sc as plsc

N_BINS = 18
B = 32
T = 16384
TC_CHUNK = 2048
N_CHUNKS = T // TC_CHUNK
ACC = (N_BINS + 2) * B

_INV_DELTA = np.float32((N_BINS) / (2.0 * np.pi))
_PI = np.float32(np.pi)


def _sc_body(pha_hbm, amp_hbm, out_hbm, pha_v, amp_v, offs_v, acc_v):
    j = lax.axis_index("s") * 2 + lax.axis_index("c")

    def zero(k, _):
        acc_v[pl.ds(k * 16, 16)] = jnp.zeros((16,), jnp.float32)
        return 0

    lax.fori_loop(0, ACC // 16, zero, 0, unroll=True)

    def chunk_body(c, _):
        pltpu.sync_copy(pha_hbm.at[pl.ds(j * T + c * TC_CHUNK, TC_CHUNK)], pha_v)
        pltpu.sync_copy(amp_hbm.at[:, pl.ds(c * TC_CHUNK, TC_CHUNK)], amp_v)

        def binify(k, _):
            v = pha_v[pl.ds(k * 16, 16)]
            f = (v + _PI) * _INV_DELTA
            idx = f.astype(jnp.int32)
            idx = jnp.minimum(idx, N_BINS)
            idx = jnp.where(f < 0.0, N_BINS, idx)
            offs_v[pl.ds(k * 16, 16)] = idx * B
            return 0

        lax.fori_loop(0, TC_CHUNK // 16, binify, 0, unroll=4)

        def accum(g, _):
            off_vec = offs_v[pl.ds(g * 16, 16)]
            for i in range(B):
                vals = amp_v[i, pl.ds(g * 16, 16)]
                plsc.addupdate_scatter(acc_v, [off_vec + i], vals)
            return 0

        lax.fori_loop(0, TC_CHUNK // 16, accum, 0)
        return 0

    lax.fori_loop(0, N_CHUNKS, chunk_body, 0)
    pltpu.sync_copy(acc_v, out_hbm.at[pl.ds(j * ACC, ACC)])


@functools.partial(
    pl.kernel,
    out_type=jax.ShapeDtypeStruct((B * ACC,), jnp.float32),
    mesh=plsc.VectorSubcoreMesh(
        core_axis_name="c", subcore_axis_name="s", num_cores=2, num_subcores=16
    ),
    scratch_types=[
        pltpu.VMEM((TC_CHUNK,), jnp.float32),
        pltpu.VMEM((B, TC_CHUNK), jnp.float32),
        pltpu.VMEM((TC_CHUNK,), jnp.int32),
        pltpu.VMEM((ACC,), jnp.float32),
    ],
    compiler_params=pltpu.CompilerParams(needs_layout_passes=False),
)
def _sc_binsum(pha_hbm, amp_hbm, out_hbm, pha_v, amp_v, offs_v, acc_v):
    _sc_body(pha_hbm, amp_hbm, out_hbm, pha_v, amp_v, offs_v, acc_v)


def _entropy_body(sums_ref, out_ref):
    s = sums_ref[...]
    s18 = s[:, :N_BINS, :]
    tot = jnp.sum(s18, axis=1, keepdims=True)
    p = s18 / tot
    inv_log_n = np.float32(1.0 / np.log(float(N_BINS)))
    mi = 1.0 + inv_log_n * jnp.sum(p * jnp.log(p), axis=1)
    out_ref[...] = mi


@jax.jit
def kernel(pha, amp):
    pha_flat = pha.reshape(-1)
    sums = _sc_binsum(pha_flat, amp)
    mit = pl.pallas_call(
        _entropy_body,
        out_shape=jax.ShapeDtypeStruct((B, B), jnp.float32),
    )(sums.reshape(B, N_BINS + 2, B))
    return mit.T


# speedup vs baseline: 4.5394x; 4.5394x over previous
import functools
import numpy as np
import jax
import jax.numpy as jnp
from jax import lax
from jax.experimental import pallas as pl
from jax.experimental.pallas import tpu as pltpu
from jax.experimental.pallas import tpu_sc as plsc

N_BINS = 18
B = 32
T = 16384
TC_CHUNK = 1024
N_CHUNKS = T // TC_CHUNK
NBINS_PAD = N_BINS + 2
ACC = NBINS_PAD * B

_INV_DELTA = np.float32(N_BINS / (2.0 * np.pi))
_PI = np.float32(np.pi)


def _binify_body(pha_ref, offs_ref):
    f = (pha_ref[...] + _PI) * _INV_DELTA
    idx = f.astype(jnp.int32)
    idx = jnp.minimum(idx, N_BINS)
    idx = jnp.where(f < 0.0, N_BINS, idx)
    offs_ref[...] = idx * B


def _sc_body(offs_hbm, ampT_hbm, out_hbm, amp_v, offs_v, acc_v, sa0, sa1, so0, so1):
    j = lax.axis_index("s") * 2 + lax.axis_index("c")
    sems_a = (sa0, sa1)
    sems_o = (so0, so1)

    def zero(k, _):
        acc_v[pl.ds(k * 16, 16)] = jnp.zeros((16,), jnp.float32)
        return 0

    lax.fori_loop(0, ACC // 16, zero, 0, unroll=True)

    def issue(c, b):
        pltpu.async_copy(
            ampT_hbm.at[pl.ds(c * TC_CHUNK * B, TC_CHUNK * B)], amp_v.at[b], sems_a[b]
        )
        pltpu.async_copy(
            offs_hbm.at[pl.ds(j * T + c * TC_CHUNK, TC_CHUNK)], offs_v.at[b], sems_o[b]
        )

    def wait(b):
        pltpu.make_async_copy(
            ampT_hbm.at[pl.ds(0, TC_CHUNK * B)], amp_v.at[b], sems_a[b]
        ).wait()
        pltpu.make_async_copy(
            offs_hbm.at[pl.ds(0, TC_CHUNK)], offs_v.at[b], sems_o[b]
        ).wait()

    def accum(b):
        @plsc.parallel_loop(0, TC_CHUNK // 16, unroll=4)
        def _(g):
            off_vec = offs_v[b, pl.ds(g * 16, 16)]
            base = g * (16 * B)
            for u in range(16):
                off = off_vec[u]
                a0 = amp_v[b, pl.ds(base + u * B, 16)]
                a1 = amp_v[b, pl.ds(base + u * B + 16, 16)]
                plsc.addupdate(acc_v.at[pl.ds(off, 16)], a0)
                plsc.addupdate(acc_v.at[pl.ds(off + 16, 16)], a1)

    issue(0, 0)
    issue(1, 1)

    def pair_body(p, _):
        c = p * 2
        wait(0)
        accum(0)

        @pl.when(p < N_CHUNKS // 2 - 1)
        def _():
            issue(c + 2, 0)

        wait(1)
        accum(1)

        @pl.when(p < N_CHUNKS // 2 - 1)
        def _():
            issue(c + 3, 1)

        return 0

    lax.fori_loop(0, N_CHUNKS // 2, pair_body, 0)
    pltpu.sync_copy(acc_v, out_hbm.at[pl.ds(j * ACC, ACC)])


@functools.partial(
    pl.kernel,
    out_type=jax.ShapeDtypeStruct((B * ACC,), jnp.float32),
    mesh=plsc.VectorSubcoreMesh(
        core_axis_name="c", subcore_axis_name="s", num_cores=2, num_subcores=16
    ),
    scratch_types=[
        pltpu.VMEM((2, TC_CHUNK * B), jnp.float32),
        pltpu.VMEM((2, TC_CHUNK), jnp.int32),
        pltpu.VMEM((ACC,), jnp.float32),
        pltpu.SemaphoreType.DMA,
        pltpu.SemaphoreType.DMA,
        pltpu.SemaphoreType.DMA,
        pltpu.SemaphoreType.DMA,
    ],
    compiler_params=pltpu.CompilerParams(needs_layout_passes=False),
)
def _sc_binsum(offs_hbm, ampT_hbm, out_hbm, amp_v, offs_v, acc_v, sa0, sa1, so0, so1):
    _sc_body(offs_hbm, ampT_hbm, out_hbm, amp_v, offs_v, acc_v, sa0, sa1, so0, so1)


def _entropy_body(sums_ref, out_ref):
    s = sums_ref[...]
    s18 = s[:, :N_BINS, :]
    tot = jnp.sum(s18, axis=1, keepdims=True)
    p = s18 / tot
    inv_log_n = np.float32(1.0 / np.log(float(N_BINS)))
    mi = 1.0 + inv_log_n * jnp.sum(p * jnp.log(p), axis=1)
    out_ref[...] = mi


@jax.jit
def kernel(pha, amp):
    offs = pl.pallas_call(
        _binify_body,
        out_shape=jax.ShapeDtypeStruct((B, T), jnp.int32),
    )(pha)
    sums = _sc_binsum(offs.reshape(-1), amp.T.reshape(-1))
    mit = pl.pallas_call(
        _entropy_body,
        out_shape=jax.ShapeDtypeStruct((B, B), jnp.float32),
    )(sums.reshape(B, NBINS_PAD, B))
    return mit.T


# speedup vs baseline: 4.8992x; 1.0792x over previous
import functools
import numpy as np
import jax
import jax.numpy as jnp
from jax import lax
from jax.experimental import pallas as pl
from jax.experimental.pallas import tpu as pltpu
from jax.experimental.pallas import tpu_sc as plsc

N_BINS = 18
B = 32
T = 16384
NW = 32
TW = T // NW
NBINS_PAD = N_BINS + 2
ACC = NBINS_PAD * B

_INV_DELTA = np.float32(N_BINS / (2.0 * np.pi))
_PI = np.float32(np.pi)


def _binify_body(pha_ref, offs_ref):
    f = (pha_ref[...] + _PI) * _INV_DELTA
    idx = f.astype(jnp.int32)
    idx = jnp.minimum(idx, N_BINS)
    idx = jnp.where(f < 0.0, N_BINS, idx)
    offs_ref[...] = idx * B


def _sc_body(offs_hbm, ampT_hbm, out_hbm, offs_v, amp_v, acc_v, sem):
    w = lax.axis_index("s") * 2 + lax.axis_index("c")

    pltpu.async_copy(
        offs_hbm.at[pl.ds(w * (B * TW), B * TW)], offs_v, sem
    )

    def zero(k, _):
        acc_v[pl.ds(k * 16, 16)] = jnp.zeros((16,), jnp.float32)
        return 0

    lax.fori_loop(0, B * ACC // 16, zero, 0)

    pltpu.make_async_copy(
        offs_hbm.at[pl.ds(0, B * TW)], offs_v, sem
    ).wait()
    pltpu.sync_copy(ampT_hbm.at[pl.ds(w * (TW * B), TW * B)], amp_v)

    def per_j(jj, _):
        obase = jj * TW
        abase = jj * ACC

        @plsc.parallel_loop(0, TW // 16, unroll=4)
        def _(g):
            off_vec = offs_v[pl.ds(obase + g * 16, 16)]
            tbase = g * (16 * B)
            for u in range(16):
                off = off_vec[u] + abase
                a0 = amp_v[pl.ds(tbase + u * B, 16)]
                a1 = amp_v[pl.ds(tbase + u * B + 16, 16)]
                plsc.addupdate(acc_v.at[pl.ds(off, 16)], a0)
                plsc.addupdate(acc_v.at[pl.ds(off + 16, 16)], a1)

        return 0

    lax.fori_loop(0, B, per_j, 0)
    pltpu.sync_copy(acc_v, out_hbm.at[pl.ds(w * (B * ACC), B * ACC)])


@functools.partial(
    pl.kernel,
    out_type=jax.ShapeDtypeStruct((NW * B * ACC,), jnp.float32),
    mesh=plsc.VectorSubcoreMesh(
        core_axis_name="c", subcore_axis_name="s", num_cores=2, num_subcores=16
    ),
    scratch_types=[
        pltpu.VMEM((B * TW,), jnp.int32),
        pltpu.VMEM((TW * B,), jnp.float32),
        pltpu.VMEM((B * ACC,), jnp.float32),
        pltpu.SemaphoreType.DMA,
    ],
    compiler_params=pltpu.CompilerParams(needs_layout_passes=False),
)
def _sc_binsum(offs_hbm, ampT_hbm, out_hbm, offs_v, amp_v, acc_v, sem):
    _sc_body(offs_hbm, ampT_hbm, out_hbm, offs_v, amp_v, acc_v, sem)


def _entropy_body(sums_ref, out_ref):
    s = sums_ref[...]
    r = jnp.sum(s, axis=0)
    s18 = r[:, :N_BINS, :]
    tot = jnp.sum(s18, axis=1, keepdims=True)
    p = s18 / tot
    inv_log_n = np.float32(1.0 / np.log(float(N_BINS)))
    mi = 1.0 + inv_log_n * jnp.sum(p * jnp.log(p), axis=1)
    out_ref[...] = mi


@jax.jit
def kernel(pha, amp):
    offs = pl.pallas_call(
        _binify_body,
        out_shape=jax.ShapeDtypeStruct((B, T), jnp.int32),
    )(pha)
    offs_t = offs.reshape(B, NW, TW).swapaxes(0, 1).reshape(-1)
    sums = _sc_binsum(offs_t, amp.T.reshape(-1))
    mit = pl.pallas_call(
        _entropy_body,
        out_shape=jax.ShapeDtypeStruct((B, B), jnp.float32),
    )(sums.reshape(NW, B, NBINS_PAD, B))
    return mit.T
